# baseline (speedup 1.0000x reference)
import numpy as np

import jax
import jax.numpy as jnp
from jax.experimental import pallas as pl

_D = 263
_B = 4
_T0 = 64
_WIDTH = 512
_CODE_DIM = 32
_NB_CODE = 256
_DEPTH = 3
_DOWN_T = 3
_DGR = 3

_MEAN_UPPER = np.asarray([0.1216, 0.2488, 0.2967, 0.5027, 0.4053, 0.41,
                          0.5703, 0.403, 0.4078, 0.1994, 0.1992, 0.0661,
                          0.0639], dtype=np.float32)
_STD_UPPER = np.asarray([0.0164, 0.0412, 0.0523, 0.0864, 0.0695, 0.0703,
                         0.1108, 0.0853, 0.0847, 0.1289, 0.1291, 0.2463,
                         0.2484], dtype=np.float32)
_SPINE_IDX = np.arange(0, 60)
_LA_IDX = np.arange(60, 108)
_RA_IDX = np.arange(101, 149)
_LL_IDX = np.arange(149, 208)
_RL_IDX = np.concatenate([np.arange(149, 153), np.arange(208, 263)])
_LOWER_MAP = np.array([0, 1, 2, 3])
_OVERLAP_LOWER_IDX = np.arange(149, 153)
_UPPER_Y_IDX = np.array([60 + 4 * i for i in range(13)])

_PARTS = ("left_arm", "right_arm", "right_leg", "left_leg", "spine")
_PART_IDX = {"left_arm": _LA_IDX, "right_arm": _RA_IDX, "right_leg": _RL_IDX,
             "left_leg": _LL_IDX, "spine": _SPINE_IDX}
_PART_DIM = {"left_arm": 48, "right_arm": 48, "right_leg": 59,
             "left_leg": 59, "spine": 60}


def _shift(x3, s):
    b, t, c = x3.shape
    if s == 0:
        return x3
    z = jnp.zeros((b, min(abs(s), t), c), dtype=x3.dtype)
    if abs(s) >= t:
        return z
    if s > 0:
        return jnp.concatenate([x3[:, s:, :], z], axis=1)
    return jnp.concatenate([z, x3[:, :s, :]], axis=1)


def _mm(a2, w_io):
    return jax.lax.dot_general(a2, w_io, (((1,), (0,)), ((), ())),
                               preferred_element_type=jnp.float32)


def _mmT(a2, w_oi):
    return jax.lax.dot_general(a2, w_oi, (((1,), (1,)), ((), ())),
                               preferred_element_type=jnp.float32)


def _taps(w2, k):
    o = w2.shape[0]
    wt = w2.T.reshape(w2.shape[1] // k, k, o)
    return [wt[:, j, :] for j in range(k)]


def _conv(x3, offsets, w2, bias):
    b, t, c = x3.shape
    taps = _taps(w2, len(offsets))
    acc = None
    for off, tap in zip(offsets, taps):
        if abs(off) >= t:
            continue
        y = _mm(_shift(x3, off).reshape(b * t, c), tap)
        acc = y if acc is None else acc + y
    return (acc + bias[None, :]).reshape(b, t, -1)


def _down_conv(x3, w2, bias):
    b, t, c = x3.shape
    to = t // 2
    taps = _taps(w2, 4)
    acc = None
    for k in range(4):
        xs = _shift(x3, k - 1).reshape(b, to, 2, c)[:, :, 0, :]
        y = _mm(xs.reshape(b * to, c), taps[k])
        acc = y if acc is None else acc + y
    return (acc + bias[None, :]).reshape(b, to, -1)


def _res_block(x3, w1, b1, w2, b2, d):
    h = jax.nn.relu(x3)
    h = _conv(h, (-d, 0, d), w1, b1)
    h = jax.nn.relu(h)
    b, t, c = h.shape
    y = _mmT(h.reshape(b * t, c), w2) + b2[None, :]
    return x3 + y.reshape(b, t, -1)


def _enc_res_chain(h, r):
    for j in range(_DEPTH):
        w1, b1, w2, b2 = r[4 * j: 4 * j + 4]
        h = _res_block(h, w1[...], b1[0], w2[...], b2[0], _DGR ** j)
    return h


def _enc_first_kernel(*refs):
    x_ref, win_ref, bin_ref, dw, db = refs[0:5]
    h_ref = refs[17]
    h = jax.nn.relu(_conv(x_ref[...], (-1, 0, 1), win_ref[...], bin_ref[0]))
    h_ref[...] = _enc_res_chain(_down_conv(h, dw[...], db[0]), refs[5:17])


def _enc_mid_kernel(*refs):
    h_ref, dw, db = refs[0:3]
    out_ref = refs[15]
    out_ref[...] = _enc_res_chain(_down_conv(h_ref[...], dw[...], db[0]),
                                  refs[3:15])


def _enc_last_kernel(*refs):
    h_ref, dw, db = refs[0:3]
    wout_ref, bout_ref, cb_ref, cbt_ref = refs[15:19]
    q_ref, stats_ref = refs[19:21]
    h = _enc_res_chain(_down_conv(h_ref[...], dw[...], db[0]), refs[3:15])
    e = _conv(h, (-1, 0, 1), wout_ref[...], bout_ref[0])
    n = _B * 8
    xf = e.reshape(n, _CODE_DIM)
    cb = cb_ref[...]
    dist = (jnp.sum(xf * xf, axis=1, keepdims=True)
            - 2.0 * _mmT(xf, cb)
            + jnp.sum(cb * cb, axis=1)[None, :])
    dmin = jnp.min(dist, axis=1, keepdims=True)
    lane = jax.lax.broadcasted_iota(jnp.int32, (n, _NB_CODE), 1)
    idx = jnp.min(jnp.where(dist <= dmin, lane, _NB_CODE), axis=1)
    onehot = (lane == idx[:, None]).astype(jnp.float32)
    xd = _mmT(onehot, cbt_ref[...])
    loss = jnp.mean((xf - xd) ** 2)
    pr = jnp.mean(onehot, axis=0)
    perp = jnp.exp(-jnp.sum(pr * jnp.log(pr + 1e-10)))
    q_ref[...] = xd.reshape(_B, 8, _CODE_DIM)
    row = jax.lax.broadcasted_iota(jnp.int32, (8, 128), 0)
    stats_ref[...] = jnp.where(row == 0, loss, jnp.where(row == 1, perp, 0.0))


def _dec_level(h, refs):
    for j in range(_DEPTH):
        w1, b1, w2, b2 = refs[4 * j: 4 * j + 4]
        h = _res_block(h, w1[...], b1[0], w2[...], b2[0],
                       _DGR ** (_DEPTH - 1 - j))
    b, t, c = h.shape
    h = jnp.broadcast_to(h[:, :, None, :], (b, t, 2, c)).reshape(b, 2 * t, c)
    return _conv(h, (-1, 0, 1), refs[12][...], refs[13][0])


def _dec_first_kernel(*refs):
    q_ref, win_ref, bin_ref = refs[0:3]
    out_ref = refs[17]
    h = jax.nn.relu(_conv(q_ref[...], (-1, 0, 1), win_ref[...], bin_ref[0]))
    out_ref[...] = _dec_level(h, refs[3:17])


def _dec_mid_kernel(*refs):
    out_ref = refs[15]
    out_ref[...] = _dec_level(refs[0][...], refs[1:15])


def _dec_last_kernel(*refs):
    wmid_ref, bmid_ref, wout_ref, bout_ref = refs[15:19]
    y_ref = refs[19]
    h = _dec_level(refs[0][...], refs[1:15])
    h = jax.nn.relu(_conv(h, (-1, 0, 1), wmid_ref[...], bmid_ref[0]))
    y_ref[...] = _conv(h, (-1, 0, 1), wout_ref[...], bout_ref[0])


def _r2(w):
    return jnp.reshape(w, (w.shape[0], -1))


def _b2(b):
    return b[None, :]


def _enc_lev_args(blk):
    a = [_r2(blk["w"]), _b2(blk["b"])]
    for rb in blk["res"]:
        a += [_r2(rb["w1"]), _b2(rb["b1"]), _r2(rb["w2"]), _b2(rb["b2"])]
    return a


def _dec_lev_args(blk):
    a = []
    for rb in blk["res"]:
        a += [_r2(rb["w1"]), _b2(rb["b1"]), _r2(rb["w2"]), _b2(rb["b2"])]
    a += [_r2(blk["w"]), _b2(blk["b"])]
    return a


def _shift_upper_down(x):
    shift_y = x[:, :, 3:4]
    upper = (x[:, :, _UPPER_Y_IDX] - shift_y - _MEAN_UPPER) / _STD_UPPER
    return x.at[:, :, _UPPER_Y_IDX].set(upper)


def _shift_upper_up(x):
    upper = x[:, :, _UPPER_Y_IDX] * _STD_UPPER + _MEAN_UPPER
    x = x.at[:, :, _UPPER_Y_IDX].set(upper)
    shift_y = x[:, :, 3:4]
    return x.at[:, :, _UPPER_Y_IDX].add(shift_y)


def _merge(la, ra, rl, ll, sp):
    motion = jnp.zeros((_B, _T0, _D), dtype=la.dtype)
    motion = motion.at[:, :, _LA_IDX].set(la)
    motion = motion.at[:, :, _RA_IDX].set(ra)
    motion = motion.at[:, :, _RL_IDX].set(rl)
    motion = motion.at[:, :, _LL_IDX].set(ll)
    motion = motion.at[:, :, _SPINE_IDX].set(sp)
    return motion.at[:, :, _OVERLAP_LOWER_IDX].set(
        (ll[:, :, _LOWER_MAP] + rl[:, :, _LOWER_MAP]) / 2.0)


def kernel(x, params):
    x = x.astype(jnp.float32)
    xs = _shift_upper_down(x)
    f32 = jnp.float32

    def hs(t):
        return jax.ShapeDtypeStruct((_B, t, _WIDTH), f32)

    qs, stats, ys = [], [], []
    for name in _PARTS:
        enc = params["enc"][name]
        cb = params["cb"][name]
        h = pl.pallas_call(_enc_first_kernel, out_shape=hs(32))(
            xs[:, :, _PART_IDX[name]], _r2(enc["w_in"]), _b2(enc["b_in"]),
            *_enc_lev_args(enc["down"][0]))
        h = pl.pallas_call(_enc_mid_kernel, out_shape=hs(16))(
            h, *_enc_lev_args(enc["down"][1]))
        q, st = pl.pallas_call(
            _enc_last_kernel,
            out_shape=[jax.ShapeDtypeStruct((_B, 8, _CODE_DIM), f32),
                       jax.ShapeDtypeStruct((8, 128), f32)],
        )(h, *_enc_lev_args(enc["down"][2]), _r2(enc["w_out"]),
          _b2(enc["b_out"]), cb, cb.T)
        qs.append(q)
        stats.append(st)
    for i, name in enumerate(_PARTS):
        dec = params["dec"][name]
        h = pl.pallas_call(_dec_first_kernel, out_shape=hs(16))(
            qs[i], _r2(dec["w_in"]), _b2(dec["b_in"]),
            *_dec_lev_args(dec["up"][0]))
        h = pl.pallas_call(_dec_mid_kernel, out_shape=hs(32))(
            h, *_dec_lev_args(dec["up"][1]))
        y = pl.pallas_call(
            _dec_last_kernel,
            out_shape=jax.ShapeDtypeStruct((_B, _T0, _PART_DIM[name]), f32),
        )(h, *_dec_lev_args(dec["up"][2]), _r2(dec["w_mid"]),
          _b2(dec["b_mid"]), _r2(dec["w_out"]), _b2(dec["b_out"]))
        ys.append(y)

    motion = _shift_upper_up(_merge(ys[0], ys[1], ys[2], ys[3], ys[4]))
    loss = sum(st[0, 0] for st in stats)
    perplexity = stats[4][1, 0]
    return motion, loss, perplexity
